# baseline (speedup 1.0000x reference)
import functools

import jax
import jax.numpy as jnp
from jax import lax
from jax.experimental import pallas as pl
from jax.experimental.pallas import tpu as pltpu
from jax.experimental.pallas import tpu_sc as plsc

VOCAB = 1000000
D_OUT = 32
BATCH = 16384

NC = 2
NS = 16
NW = NC * NS
B_PER_W = BATCH // NW
LANES = 16
NBUF = 4
TAIL = 999936
LAST_COL = 999808


def _build():
    mesh = plsc.VectorSubcoreMesh(core_axis_name="c", subcore_axis_name="s")

    @functools.partial(
        pl.kernel,
        mesh=mesh,
        compiler_params=pltpu.CompilerParams(
            use_tc_tiling_on_sc=True, needs_layout_passes=False
        ),
        out_type=jax.ShapeDtypeStruct((4, 8, BATCH), jnp.float32),
        scratch_types=[
            pltpu.VMEM((B_PER_W,), jnp.int32),
            pltpu.VMEM((4, 8, 64), jnp.float32),
            [pltpu.VMEM((4, 8, 128), jnp.float32) for _ in range(NBUF)],
            pltpu.VMEM((4, 8, B_PER_W), jnp.float32),
            [pltpu.SemaphoreType.DMA for _ in range(NBUF)],
            pltpu.SemaphoreType.DMA,
        ],
    )
    def gather_kernel(
        idx_hbm, tab3_hbm, out4_hbm, idx_v, tail_v, bufs, colbuf, sems, sem
    ):
        wid = lax.axis_index("s") * NC + lax.axis_index("c")
        base = wid * B_PER_W
        pltpu.sync_copy(idx_hbm.at[pl.ds(base, B_PER_W)], idx_v)
        pltpu.sync_copy(tab3_hbm.at[:, :, pl.ds(TAIL, 64)], tail_v)

        lane = lax.iota(jnp.int32, LANES)
        ct_lo, cs_lo = lane >> 3, lane & 7
        ct_hi, cs_hi = ct_lo + 2, cs_lo

        def read_idx(i):
            vec = idx_v[pl.ds((i >> 4) << 4, LANES)]
            return lax.reduce_sum(
                jnp.where(lane == (i & 15), vec, 0), axes=(0,)
            )

        def col_of(r):
            c = jnp.minimum((r >> 7) << 7, LAST_COL)
            return pl.multiple_of(c, 128)

        def fetch(i, b):
            r = read_idx(i)
            col = col_of(r)
            for ct in range(4):
                pltpu.async_copy(
                    tab3_hbm.at[ct, :, pl.ds(col, 128)],
                    bufs[b].at[ct],
                    sems[b],
                )

        def process(i, b):
            r = read_idx(i)
            pltpu.make_async_copy(
                tab3_hbm.at[:, :, pl.ds(0, 128)], bufs[b], sems[b]
            ).wait()
            slot = jnp.broadcast_to(i, (LANES,))

            @pl.when(r < TAIL)
            def _():
                rel = jnp.broadcast_to(r - col_of(r), (LANES,))
                for ct, cs in ((ct_lo, cs_lo), (ct_hi, cs_hi)):
                    vals = plsc.load_gather(bufs[b], [ct, cs, rel])
                    plsc.store_scatter(colbuf, [ct, cs, slot], vals)

            @pl.when(r >= TAIL)
            def _():
                trel = jnp.broadcast_to(r - TAIL, (LANES,))
                for ct, cs in ((ct_lo, cs_lo), (ct_hi, cs_hi)):
                    vals = plsc.load_gather(tail_v, [ct, cs, trel])
                    plsc.store_scatter(colbuf, [ct, cs, slot], vals)

        for b in range(NBUF):
            fetch(b, b)

        def step(g, _):
            i0 = g * NBUF
            for b in range(NBUF):
                process(i0 + b, b)
                fetch(i0 + b + NBUF, b)
            return ()

        lax.fori_loop(0, B_PER_W // NBUF - 1, step, (), unroll=False)
        for b in range(NBUF):
            process(B_PER_W - NBUF + b, b)

        pltpu.sync_copy(colbuf, out4_hbm.at[:, :, pl.ds(base, B_PER_W)])

    return gather_kernel


_GATHER = _build()


@jax.jit
def kernel(x, eps_table):
    tab3 = eps_table.T.reshape(4, 8, VOCAB)
    out4 = _GATHER(x, tab3)
    return out4.reshape(D_OUT, BATCH).T


# speedup vs baseline: 1.2769x; 1.2769x over previous
import functools

import jax
import jax.numpy as jnp
from jax import lax
from jax.experimental import pallas as pl
from jax.experimental.pallas import tpu as pltpu
from jax.experimental.pallas import tpu_sc as plsc

VOCAB = 1000000
D_OUT = 32
BATCH = 16384

NC = 2
NS = 16
NW = NC * NS
B_PER_W = BATCH // NW
LANES = 16
NBUF = 8
TAIL = 999936
LAST_COL = 999808


def _build():
    mesh = plsc.VectorSubcoreMesh(core_axis_name="c", subcore_axis_name="s")

    @functools.partial(
        pl.kernel,
        mesh=mesh,
        compiler_params=pltpu.CompilerParams(
            use_tc_tiling_on_sc=True, needs_layout_passes=False
        ),
        out_type=jax.ShapeDtypeStruct((4, 8, BATCH), jnp.float32),
        scratch_types=[
            pltpu.VMEM((B_PER_W,), jnp.int32),
            pltpu.VMEM((4, 8, 64), jnp.float32),
            [pltpu.VMEM((4, 8, 128), jnp.float32) for _ in range(NBUF)],
            pltpu.VMEM((4, 8, B_PER_W), jnp.float32),
            [pltpu.SemaphoreType.DMA for _ in range(NBUF)],
            pltpu.SemaphoreType.DMA,
        ],
    )
    def gather_kernel(
        idx_hbm, tab3_hbm, out4_hbm, idx_v, tail_v, bufs, colbuf, sems, sem
    ):
        wid = lax.axis_index("s") * NC + lax.axis_index("c")
        base = wid * B_PER_W
        pltpu.sync_copy(idx_hbm.at[pl.ds(base, B_PER_W)], idx_v)
        pltpu.sync_copy(tab3_hbm.at[:, :, pl.ds(TAIL, 64)], tail_v)

        lane = lax.iota(jnp.int32, LANES)
        ct_lo, cs_lo = lane >> 3, lane & 7
        ct_hi, cs_hi = ct_lo + 2, cs_lo

        def read_idx(i):
            vec = idx_v[pl.ds((i >> 4) << 4, LANES)]
            return lax.reduce_sum(
                jnp.where(lane == (i & 15), vec, 0), axes=(0,)
            )

        def col_of(r):
            c = jnp.minimum((r >> 7) << 7, LAST_COL)
            return pl.multiple_of(c, 128)

        def fetch(i, b):
            r = read_idx(i)
            col = col_of(r)
            for ct in range(4):
                pltpu.async_copy(
                    tab3_hbm.at[ct, :, pl.ds(col, 128)],
                    bufs[b].at[ct],
                    sems[b],
                )

        def process(i, b):
            r = read_idx(i)
            pltpu.make_async_copy(
                tab3_hbm.at[:, :, pl.ds(0, 128)], bufs[b], sems[b]
            ).wait()
            slot = jnp.broadcast_to(i, (LANES,))

            @pl.when(r < TAIL)
            def _():
                rel = jnp.broadcast_to(r - col_of(r), (LANES,))
                for ct, cs in ((ct_lo, cs_lo), (ct_hi, cs_hi)):
                    vals = plsc.load_gather(bufs[b], [ct, cs, rel])
                    plsc.store_scatter(colbuf, [ct, cs, slot], vals)

            @pl.when(r >= TAIL)
            def _():
                trel = jnp.broadcast_to(r - TAIL, (LANES,))
                for ct, cs in ((ct_lo, cs_lo), (ct_hi, cs_hi)):
                    vals = plsc.load_gather(tail_v, [ct, cs, trel])
                    plsc.store_scatter(colbuf, [ct, cs, slot], vals)

        for b in range(NBUF):
            fetch(b, b)

        def step(g, _):
            i0 = g * NBUF
            for b in range(NBUF):
                process(i0 + b, b)
                fetch(i0 + b + NBUF, b)
            return ()

        lax.fori_loop(0, B_PER_W // NBUF - 1, step, (), unroll=False)
        for b in range(NBUF):
            process(B_PER_W - NBUF + b, b)

        pltpu.sync_copy(colbuf, out4_hbm.at[:, :, pl.ds(base, B_PER_W)])

    return gather_kernel


_GATHER = _build()


@jax.jit
def kernel(x, eps_table):
    tab3 = eps_table.T.reshape(4, 8, VOCAB)
    out4 = _GATHER(x, tab3)
    return out4.reshape(D_OUT, BATCH).T
